# Initial kernel scaffold; baseline (speedup 1.0000x reference)
import jax
import jax.numpy as jnp
from jax.experimental import pallas as pl


def kernel(x, edge_index, W1, b1, W2, b2, W3, b3, Wc, bc):
    raise NotImplementedError("write your pallas kernel here")


# speedup vs baseline: 43.8750x; 43.8750x over previous
import functools

import jax
import jax.numpy as jnp
from jax import lax
from jax.experimental import pallas as pl
from jax.experimental.pallas import tpu as pltpu
from jax.experimental.pallas import tpu_sc as plsc

N = 10000
E = 320000
NC = 2
NS = 16
L = 16
NW = NC * NS
EPW = E // NW
NPAD = 10240


RCH = 128


def _make_agg(F):
  W = NPAD * F
  ROWS = W // 8
  NCH = ROWS // RCH
  RPT = ROWS // NS
  mesh = plsc.VectorSubcoreMesh(core_axis_name="c", subcore_axis_name="s")

  @functools.partial(
      pl.kernel,
      out_type=jax.ShapeDtypeStruct((NC, ROWS, 8), jnp.float32),
      mesh=mesh,
      compiler_params=pltpu.CompilerParams(needs_layout_passes=False,
                                           use_tc_tiling_on_sc=False),
      scratch_types=[
          pltpu.VMEM((W,), jnp.float32),
          pltpu.VMEM((ROWS, 8), jnp.float32),
          pltpu.VMEM((EPW,), jnp.int32),
          pltpu.VMEM((EPW,), jnp.int32),
          pltpu.VMEM((NCH, RCH), jnp.int32),
          pltpu.VMEM_SHARED((ROWS, 8), jnp.float32),
          pltpu.VMEM_SHARED((W,), jnp.float32),
      ],
  )
  def agg(y_hbm, zeros_hbm, idx_hbm, src_hbm, dst_hbm, out_hbm,
          y_v, acc_v, src_v, dst_v, idx_v, acc_sh, y_sh):
    cid = lax.axis_index("c")
    sid = lax.axis_index("s")
    wid = cid * NS + sid

    @pl.when(sid == 0)
    def _():
      pltpu.sync_copy(y_hbm, y_sh)
      pltpu.sync_copy(zeros_hbm, acc_sh)

    plsc.subcore_barrier()
    pltpu.sync_copy(y_sh, y_v)
    pltpu.sync_copy(acc_sh, acc_v)
    pltpu.sync_copy(idx_hbm, idx_v)
    base = wid * EPW
    pltpu.sync_copy(src_hbm.at[pl.ds(base, EPW)], src_v)
    pltpu.sync_copy(dst_hbm.at[pl.ds(base, EPW)], dst_v)
    plsc.subcore_barrier()

    def eb(i, c):
      s = src_v[pl.ds(i * L, L)] * F
      d = dst_v[pl.ds(i * L, L)] * F
      for f in range(F):
        v = plsc.load_gather(y_v, [s + f])
        w = d + f
        plsc.addupdate_scatter(acc_v, [w >> 3, w & 7], v)
      return c

    lax.fori_loop(0, EPW // L, eb, 0)

    def rb(c, carry):
      pltpu.sync_copy(acc_v.at[pl.ds(c * RCH, RCH)],
                      acc_sh.at[idx_v.at[c]], add=True)
      return carry

    lax.fori_loop(0, NCH, rb, 0)
    plsc.subcore_barrier()

    r0 = sid * RPT
    pltpu.sync_copy(acc_sh.at[pl.ds(r0, RPT)], out_hbm.at[cid, pl.ds(r0, RPT)])

  return agg


_agg1 = _make_agg(1)
_agg4 = _make_agg(4)
_agg2 = _make_agg(2)


def _stage1_body(x_ref, w1_ref, degp_ref, y1_ref, dinv_ref):
  deg = degp_ref[0, :N, :] + degp_ref[1, :N, :] + 1.0
  dinv = lax.rsqrt(deg)
  xw = jnp.dot(x_ref[...], w1_ref[...], preferred_element_type=jnp.float32)
  y1_ref[:N, :] = dinv * xw
  y1_ref[N:, :] = jnp.zeros((NPAD - N, 4), jnp.float32)
  dinv_ref[...] = dinv


def _mid_body(F_out, aggp_ref, y_ref, dinv_ref, w_ref, b_ref, yn_ref):
  a = aggp_ref[0, :N, :] + aggp_ref[1, :N, :] + y_ref[:N, :]
  h = jnp.tanh(dinv_ref[...] * a + b_ref[...])
  yn = dinv_ref[...] * jnp.dot(h, w_ref[...], preferred_element_type=jnp.float32)
  yn_ref[:N, :] = yn
  yn_ref[N:, :] = jnp.zeros((NPAD - N, F_out), jnp.float32)


def _final_body(aggp_ref, y_ref, dinv_ref, b3_ref, wc_ref, bc_ref,
                out_ref, h_ref):
  a = aggp_ref[0, :N, :] + aggp_ref[1, :N, :] + y_ref[:N, :]
  h3 = jnp.tanh(dinv_ref[...] * a + b3_ref[...])
  logits = jnp.dot(h3, wc_ref[...], preferred_element_type=jnp.float32) + bc_ref[...]
  m = jnp.max(logits, axis=1, keepdims=True)
  e = logits - m
  lse = jnp.log(jnp.sum(jnp.exp(e), axis=1, keepdims=True))
  out_ref[...] = e - lse
  h_ref[...] = h3


_stage1 = pl.pallas_call(
    _stage1_body,
    out_shape=(jax.ShapeDtypeStruct((NPAD, 4), jnp.float32),
               jax.ShapeDtypeStruct((N, 1), jnp.float32)))
_mid44 = pl.pallas_call(
    functools.partial(_mid_body, 4),
    out_shape=jax.ShapeDtypeStruct((NPAD, 4), jnp.float32))
_mid42 = pl.pallas_call(
    functools.partial(_mid_body, 2),
    out_shape=jax.ShapeDtypeStruct((NPAD, 2), jnp.float32))
_final = pl.pallas_call(
    _final_body,
    out_shape=(jax.ShapeDtypeStruct((N, 4), jnp.float32),
               jax.ShapeDtypeStruct((N, 2), jnp.float32)))


@jax.jit
def kernel(x, edge_index, W1, b1, W2, b2, W3, b3, Wc, bc):
  src = edge_index[0].astype(jnp.int32)
  dst = edge_index[1].astype(jnp.int32)

  def aux(F):
    rows = NPAD * F // 8
    idx = jnp.arange(rows, dtype=jnp.int32).reshape(rows // RCH, RCH)
    return jnp.zeros((rows, 8), jnp.float32), idx

  z1, i1 = aux(1)
  z2, i2 = aux(2)
  z4, i4 = aux(4)
  ones = jnp.ones((NPAD,), jnp.float32)

  degp = _agg1(ones, z1, i1, src, dst).reshape(NC, NPAD, 1)

  y1, dinv = _stage1(x, W1, degp)

  agg1 = _agg4(y1.reshape(-1), z4, i4, src, dst).reshape(NC, NPAD, 4)
  y2 = _mid44(agg1, y1, dinv, W2, b1.reshape(1, 4))

  agg2 = _agg4(y2.reshape(-1), z4, i4, src, dst).reshape(NC, NPAD, 4)
  y3 = _mid42(agg2, y2, dinv, W3, b2.reshape(1, 4))

  agg3 = _agg2(y3.reshape(-1), z2, i2, src, dst).reshape(NC, NPAD, 2)
  out, h = _final(agg3, y3, dinv, b3.reshape(1, 2), Wc, bc.reshape(1, 4))
  return out, h


# speedup vs baseline: 50.9238x; 1.1607x over previous
import functools

import jax
import jax.numpy as jnp
from jax import lax
from jax.experimental import pallas as pl
from jax.experimental.pallas import tpu as pltpu
from jax.experimental.pallas import tpu_sc as plsc

N = 10000
E = 320000
NC = 2
NS = 16
L = 16
NW = NC * NS
EPW = E // NW
NPAD = 10240


RCH = 128


def _make_agg(F):
  W = NPAD * F
  ROWS = W // 8
  NCH = ROWS // RCH
  RPT = ROWS // NS
  mesh = plsc.VectorSubcoreMesh(core_axis_name="c", subcore_axis_name="s")

  @functools.partial(
      pl.kernel,
      out_type=jax.ShapeDtypeStruct((NC, ROWS, 8), jnp.float32),
      mesh=mesh,
      compiler_params=pltpu.CompilerParams(needs_layout_passes=False,
                                           use_tc_tiling_on_sc=False),
      scratch_types=[
          pltpu.VMEM((W,), jnp.float32),
          pltpu.VMEM((ROWS, 8), jnp.float32),
          pltpu.VMEM((EPW,), jnp.int32),
          pltpu.VMEM((EPW,), jnp.int32),
          pltpu.VMEM((NCH, RCH), jnp.int32),
          pltpu.VMEM_SHARED((ROWS, 8), jnp.float32),
          pltpu.VMEM_SHARED((W,), jnp.float32),
      ],
  )
  def agg(y_hbm, zeros_hbm, idx_hbm, src_hbm, dst_hbm, out_hbm,
          y_v, acc_v, src_v, dst_v, idx_v, acc_sh, y_sh):
    cid = lax.axis_index("c")
    sid = lax.axis_index("s")
    wid = cid * NS + sid

    @pl.when(sid == 0)
    def _():
      pltpu.sync_copy(y_hbm, y_sh)
      pltpu.sync_copy(zeros_hbm, acc_sh)

    plsc.subcore_barrier()
    pltpu.sync_copy(y_sh, y_v)
    pltpu.sync_copy(acc_sh, acc_v)
    pltpu.sync_copy(idx_hbm, idx_v)
    base = wid * EPW
    pltpu.sync_copy(src_hbm.at[pl.ds(base, EPW)], src_v)
    pltpu.sync_copy(dst_hbm.at[pl.ds(base, EPW)], dst_v)
    plsc.subcore_barrier()

    @plsc.parallel_loop(0, EPW // L, unroll=4)
    def eb(i):
      s = src_v[pl.ds(i * L, L)] * F
      d = dst_v[pl.ds(i * L, L)] * F
      for f in range(F):
        v = plsc.load_gather(y_v, [s + f])
        w = d + f
        plsc.addupdate_scatter(acc_v, [w >> 3, w & 7], v)

    def rb(c, carry):
      pltpu.sync_copy(acc_v.at[pl.ds(c * RCH, RCH)],
                      acc_sh.at[idx_v.at[c]], add=True)
      return carry

    lax.fori_loop(0, NCH, rb, 0)
    plsc.subcore_barrier()

    r0 = sid * RPT
    pltpu.sync_copy(acc_sh.at[pl.ds(r0, RPT)], out_hbm.at[cid, pl.ds(r0, RPT)])

  return agg


_agg1 = _make_agg(1)
_agg4 = _make_agg(4)
_agg2 = _make_agg(2)


def _stage1_body(x_ref, w1_ref, degp_ref, y1_ref, dinv_ref):
  deg = degp_ref[0, :N, :] + degp_ref[1, :N, :] + 1.0
  dinv = lax.rsqrt(deg)
  xw = jnp.dot(x_ref[...], w1_ref[...], preferred_element_type=jnp.float32)
  y1_ref[:N, :] = dinv * xw
  y1_ref[N:, :] = jnp.zeros((NPAD - N, 4), jnp.float32)
  dinv_ref[...] = dinv


def _mid_body(F_out, aggp_ref, y_ref, dinv_ref, w_ref, b_ref, yn_ref):
  a = aggp_ref[0, :N, :] + aggp_ref[1, :N, :] + y_ref[:N, :]
  h = jnp.tanh(dinv_ref[...] * a + b_ref[...])
  yn = dinv_ref[...] * jnp.dot(h, w_ref[...], preferred_element_type=jnp.float32)
  yn_ref[:N, :] = yn
  yn_ref[N:, :] = jnp.zeros((NPAD - N, F_out), jnp.float32)


def _final_body(aggp_ref, y_ref, dinv_ref, b3_ref, wc_ref, bc_ref,
                out_ref, h_ref):
  a = aggp_ref[0, :N, :] + aggp_ref[1, :N, :] + y_ref[:N, :]
  h3 = jnp.tanh(dinv_ref[...] * a + b3_ref[...])
  logits = jnp.dot(h3, wc_ref[...], preferred_element_type=jnp.float32) + bc_ref[...]
  m = jnp.max(logits, axis=1, keepdims=True)
  e = logits - m
  lse = jnp.log(jnp.sum(jnp.exp(e), axis=1, keepdims=True))
  out_ref[...] = e - lse
  h_ref[...] = h3


_stage1 = pl.pallas_call(
    _stage1_body,
    out_shape=(jax.ShapeDtypeStruct((NPAD, 4), jnp.float32),
               jax.ShapeDtypeStruct((N, 1), jnp.float32)))
_mid44 = pl.pallas_call(
    functools.partial(_mid_body, 4),
    out_shape=jax.ShapeDtypeStruct((NPAD, 4), jnp.float32))
_mid42 = pl.pallas_call(
    functools.partial(_mid_body, 2),
    out_shape=jax.ShapeDtypeStruct((NPAD, 2), jnp.float32))
_final = pl.pallas_call(
    _final_body,
    out_shape=(jax.ShapeDtypeStruct((N, 4), jnp.float32),
               jax.ShapeDtypeStruct((N, 2), jnp.float32)))


@jax.jit
def kernel(x, edge_index, W1, b1, W2, b2, W3, b3, Wc, bc):
  src = edge_index[0].astype(jnp.int32)
  dst = edge_index[1].astype(jnp.int32)

  def aux(F):
    rows = NPAD * F // 8
    idx = jnp.arange(rows, dtype=jnp.int32).reshape(rows // RCH, RCH)
    return jnp.zeros((rows, 8), jnp.float32), idx

  z1, i1 = aux(1)
  z2, i2 = aux(2)
  z4, i4 = aux(4)
  ones = jnp.ones((NPAD,), jnp.float32)

  degp = _agg1(ones, z1, i1, src, dst).reshape(NC, NPAD, 1)

  y1, dinv = _stage1(x, W1, degp)

  agg1 = _agg4(y1.reshape(-1), z4, i4, src, dst).reshape(NC, NPAD, 4)
  y2 = _mid44(agg1, y1, dinv, W2, b1.reshape(1, 4))

  agg2 = _agg4(y2.reshape(-1), z4, i4, src, dst).reshape(NC, NPAD, 4)
  y3 = _mid42(agg2, y2, dinv, W3, b2.reshape(1, 4))

  agg3 = _agg2(y3.reshape(-1), z2, i2, src, dst).reshape(NC, NPAD, 2)
  out, h = _final(agg3, y3, dinv, b3.reshape(1, 2), Wc, bc.reshape(1, 4))
  return out, h


# speedup vs baseline: 64.6493x; 1.2695x over previous
import functools

import jax
import jax.numpy as jnp
from jax import lax
from jax.experimental import pallas as pl
from jax.experimental.pallas import tpu as pltpu
from jax.experimental.pallas import tpu_sc as plsc

N = 10000
E = 320000
NS = 16
L = 16
EPT = E // NS
EPH = EPT // 2
NPAD = 10240
NSL = NPAD // NS
RCH = 128
ROWS4 = NPAD * 4 // 8
ROWS2 = NPAD * 2 // 8
ROWS1 = NPAD * 1 // 8

_mesh = plsc.VectorSubcoreMesh(core_axis_name="c", subcore_axis_name="s",
                               num_cores=1)


@functools.partial(
    pl.kernel,
    out_type=(jax.ShapeDtypeStruct((ROWS2, 8), jnp.float32),
              jax.ShapeDtypeStruct((NPAD * 2,), jnp.float32),
              jax.ShapeDtypeStruct((NPAD,), jnp.float32)),
    mesh=_mesh,
    compiler_params=pltpu.CompilerParams(needs_layout_passes=False,
                                         use_tc_tiling_on_sc=False),
    scratch_types=[
        pltpu.VMEM((NPAD * 4,), jnp.float32),
        pltpu.VMEM((ROWS4, 8), jnp.float32),
        pltpu.VMEM((EPH,), jnp.int32),
        pltpu.VMEM((EPH,), jnp.int32),
        pltpu.VMEM((ROWS4 // RCH, RCH), jnp.int32),
        pltpu.VMEM((NSL // 8, 8), jnp.float32),
        pltpu.VMEM((NSL * 4 // 8, 8), jnp.float32),
        pltpu.VMEM((NSL,), jnp.float32),
        pltpu.VMEM((NSL * 4,), jnp.float32),
        pltpu.VMEM((NSL * 4,), jnp.float32),
        pltpu.VMEM((32,), jnp.float32),
        pltpu.VMEM_SHARED((NPAD * 4,), jnp.float32),
        pltpu.VMEM_SHARED((ROWS4, 8), jnp.float32),
    ],
)
def _sc_mega(xw1_hbm, src_hbm, dst_hbm, idx_hbm, par_hbm,
             agg3_hbm, y3_hbm, dinv_hbm,
             y_v, acc_v, srcb, dstb, idx_v, deg2d, asl2d, dinv_sl,
             y_sl, h_sl, par_v, y_sh, acc_sh):
  sid = lax.axis_index("s")
  nb = sid * NSL
  ebase = sid * EPT
  iota = lax.iota(jnp.int32, L)
  ones16 = jnp.ones((L,), jnp.float32)
  zero16 = jnp.zeros((L,), jnp.float32)

  pltpu.sync_copy(idx_hbm, idx_v)
  pltpu.sync_copy(par_hbm, par_v)

  @plsc.parallel_loop(0, ROWS1 * 8 // L, unroll=4)
  def _z0(i):
    w = i * L + iota
    plsc.store_scatter(acc_v, [w >> 3, w & 7], zero16)

  @pl.when(sid == 0)
  def _():
    pltpu.sync_copy(acc_v.at[pl.ds(0, ROWS1)], acc_sh.at[pl.ds(0, ROWS1)])

  plsc.subcore_barrier()

  for half in range(2):
    pltpu.sync_copy(dst_hbm.at[pl.ds(ebase + half * EPH, EPH)], dstb)

    @plsc.parallel_loop(0, EPH // L, unroll=4)
    def _deg(i):
      d = dstb[pl.ds(i * L, L)]
      plsc.addupdate_scatter(acc_v, [d >> 3, d & 7], ones16)

  def _red1(c, carry):
    pltpu.sync_copy(acc_v.at[pl.ds(c * RCH, RCH)],
                    acc_sh.at[idx_v.at[c]], add=True)
    return carry

  lax.fori_loop(0, ROWS1 // RCH, _red1, 0)
  plsc.subcore_barrier()

  pltpu.sync_copy(acc_sh.at[pl.ds(sid * (NSL // 8), NSL // 8)], deg2d)

  @plsc.parallel_loop(0, NSL // L, unroll=2)
  def _dv(i):
    w = i * L + iota
    dg = plsc.load_gather(deg2d, [w >> 3, w & 7]) + 1.0
    ib = plsc.bitcast(dg, jnp.int32)
    ib = jnp.int32(0x5F3759DF) - (ib >> 1)
    yv = plsc.bitcast(ib, jnp.float32)
    for _ in range(3):
      yv = yv * (1.5 - 0.5 * dg * yv * yv)
    dinv_sl[pl.ds(i * L, L)] = yv

  pltpu.sync_copy(dinv_sl, dinv_hbm.at[pl.ds(nb, NSL)])

  pltpu.sync_copy(xw1_hbm.at[pl.ds(nb * 4, NSL * 4)], y_sl)

  @plsc.parallel_loop(0, NSL * 4 // L, unroll=2)
  def _y1(i):
    w = i * L + iota
    dv = plsc.load_gather(dinv_sl, [w >> 2])
    y_sl[pl.ds(i * L, L)] = y_sl[pl.ds(i * L, L)] * dv

  pltpu.sync_copy(y_sl, y_sh.at[pl.ds(sid * (NSL * 4), NSL * 4)])
  plsc.subcore_barrier()
  pltpu.sync_copy(y_sh, y_v)

  for li, F in enumerate((4, 4, 2)):
    rows = NPAD * F // 8

    @plsc.parallel_loop(0, rows * 8 // L, unroll=4)
    def _z(i):
      w = i * L + iota
      plsc.store_scatter(acc_v, [w >> 3, w & 7], zero16)

    @pl.when(sid == 0)
    def _():
      pltpu.sync_copy(acc_v.at[pl.ds(0, rows)], acc_sh.at[pl.ds(0, rows)])

    plsc.subcore_barrier()

    for half in range(2):
      pltpu.sync_copy(src_hbm.at[pl.ds(ebase + half * EPH, EPH)], srcb)
      pltpu.sync_copy(dst_hbm.at[pl.ds(ebase + half * EPH, EPH)], dstb)

      @plsc.parallel_loop(0, EPH // L, unroll=4)
      def _e(i):
        s = srcb[pl.ds(i * L, L)] * F
        d = dstb[pl.ds(i * L, L)] * F
        for f in range(F):
          v = plsc.load_gather(y_v, [s + f])
          w = d + f
          plsc.addupdate_scatter(acc_v, [w >> 3, w & 7], v)

    def _red(c, carry):
      pltpu.sync_copy(acc_v.at[pl.ds(c * RCH, RCH)],
                      acc_sh.at[idx_v.at[c]], add=True)
      return carry

    lax.fori_loop(0, rows // RCH, _red, 0)
    plsc.subcore_barrier()

    if li < 2:
      F_out = 4 if li == 0 else 2
      woff = 0 if li == 0 else 16
      boff = 24 if li == 0 else 28
      srow = NSL * F // 8
      pltpu.sync_copy(acc_sh.at[pl.ds(sid * srow, srow)],
                      asl2d.at[pl.ds(0, srow)])

      @plsc.parallel_loop(0, NSL * F // L, unroll=2)
      def _h(i):
        w = i * L + iota
        a = plsc.load_gather(asl2d, [w >> 3, w & 7])
        yv = y_sl[pl.ds(i * L, L)]
        dv = plsc.load_gather(dinv_sl, [w >> 2])
        b = plsc.load_gather(par_v, [(w & 3) + boff])
        xx = dv * (a + yv) + b
        ex = jnp.exp(2.0 * xx)
        h_sl[pl.ds(i * L, L)] = 1.0 - 2.0 / (ex + 1.0)

      sh_out = 2 if F_out == 4 else 1

      @plsc.parallel_loop(0, NSL * F_out // L, unroll=2)
      def _y(i):
        w2 = i * L + iota
        n = w2 >> sh_out
        g = w2 & (F_out - 1)
        s = zero16
        for f in range(4):
          hv = plsc.load_gather(h_sl, [n * 4 + f])
          wv = plsc.load_gather(par_v, [woff + f * F_out + g])
          s = s + hv * wv
        dv = plsc.load_gather(dinv_sl, [n])
        y_sl[pl.ds(i * L, L)] = dv * s

      slw = NSL * F_out
      pltpu.sync_copy(y_sl.at[pl.ds(0, slw)],
                      y_sh.at[pl.ds(sid * slw, slw)])
      plsc.subcore_barrier()
      pltpu.sync_copy(y_sh.at[pl.ds(0, NPAD * F_out)],
                      y_v.at[pl.ds(0, NPAD * F_out)])
    else:
      myr = rows // NS
      pltpu.sync_copy(acc_sh.at[pl.ds(sid * myr, myr)],
                      agg3_hbm.at[pl.ds(sid * myr, myr)])
      pltpu.sync_copy(y_sl.at[pl.ds(0, NSL * 2)],
                      y3_hbm.at[pl.ds(sid * (NSL * 2), NSL * 2)])


def _xw1_body(x_ref, w1_ref, o_ref):
  o_ref[:N, :] = jnp.dot(x_ref[...], w1_ref[...],
                         preferred_element_type=jnp.float32)
  o_ref[N:, :] = jnp.zeros((NPAD - N, 4), jnp.float32)


def _final_body(agg_ref, y_ref, dinv_ref, b3_ref, wc_ref, bc_ref,
                out_ref, h_ref):
  a = agg_ref[:N, :] + y_ref[:N, :]
  h3 = jnp.tanh(dinv_ref[...] * a + b3_ref[...])
  logits = jnp.dot(h3, wc_ref[...], preferred_element_type=jnp.float32) + bc_ref[...]
  m = jnp.max(logits, axis=1, keepdims=True)
  e = logits - m
  lse = jnp.log(jnp.sum(jnp.exp(e), axis=1, keepdims=True))
  out_ref[...] = e - lse
  h_ref[...] = h3


_xw1 = pl.pallas_call(
    _xw1_body, out_shape=jax.ShapeDtypeStruct((NPAD, 4), jnp.float32))
_final = pl.pallas_call(
    _final_body,
    out_shape=(jax.ShapeDtypeStruct((N, 4), jnp.float32),
               jax.ShapeDtypeStruct((N, 2), jnp.float32)))


@jax.jit
def kernel(x, edge_index, W1, b1, W2, b2, W3, b3, Wc, bc):
  src = edge_index[0].astype(jnp.int32)
  dst = edge_index[1].astype(jnp.int32)
  idx = jnp.arange(ROWS4, dtype=jnp.int32).reshape(ROWS4 // RCH, RCH)
  par = jnp.concatenate([W2.reshape(-1), W3.reshape(-1),
                         b1.reshape(-1), b2.reshape(-1)]).astype(jnp.float32)

  xw1 = _xw1(x, W1).reshape(-1)
  agg3, y3, dinv = _sc_mega(xw1, src, dst, idx, par)

  out, h = _final(agg3.reshape(NPAD, 2), y3.reshape(NPAD, 2),
                  dinv[:N].reshape(N, 1), b3.reshape(1, 2), Wc,
                  bc.reshape(1, 4))
  return out, h


# speedup vs baseline: 73.3367x; 1.1344x over previous
import functools

import jax
import jax.numpy as jnp
from jax import lax
from jax.experimental import pallas as pl
from jax.experimental.pallas import tpu as pltpu
from jax.experimental.pallas import tpu_sc as plsc

N = 10000
E = 320000
NS = 16
L = 16
EPT = E // NS
NQ = 5
EPC = EPT // NQ
NPAD = 10240
NSL = NPAD // NS
RCH = 128
ROWS4 = NPAD * 4 // 8
ROWS2 = NPAD * 2 // 8
ROWS1 = NPAD * 1 // 8

_mesh = plsc.VectorSubcoreMesh(core_axis_name="c", subcore_axis_name="s",
                               num_cores=1)


@functools.partial(
    pl.kernel,
    out_type=(jax.ShapeDtypeStruct((ROWS2, 8), jnp.float32),
              jax.ShapeDtypeStruct((NPAD * 2,), jnp.float32),
              jax.ShapeDtypeStruct((NPAD,), jnp.float32)),
    mesh=_mesh,
    compiler_params=pltpu.CompilerParams(needs_layout_passes=False,
                                         use_tc_tiling_on_sc=False),
    scratch_types=[
        pltpu.VMEM((NPAD * 4,), jnp.float32),
        pltpu.VMEM((ROWS4, 8), jnp.float32),
        pltpu.VMEM((EPC,), jnp.int32),
        pltpu.VMEM((EPC,), jnp.int32),
        pltpu.VMEM((EPC,), jnp.int32),
        pltpu.VMEM((EPC,), jnp.int32),
        pltpu.VMEM((ROWS4 // RCH, RCH), jnp.int32),
        pltpu.VMEM((NSL // 8, 8), jnp.float32),
        pltpu.VMEM((NSL * 4 // 8, 8), jnp.float32),
        pltpu.VMEM((NSL,), jnp.float32),
        pltpu.VMEM((NSL * 4,), jnp.float32),
        pltpu.VMEM((NSL * 4,), jnp.float32),
        pltpu.VMEM((32,), jnp.float32),
        pltpu.VMEM_SHARED((NPAD * 4,), jnp.float32),
        pltpu.VMEM_SHARED((ROWS4, 8), jnp.float32),
        pltpu.SemaphoreType.DMA,
        pltpu.SemaphoreType.DMA,
    ],
)
def _sc_mega(xw1_hbm, src_hbm, dst_hbm, idx_hbm, par_hbm,
             agg3_hbm, y3_hbm, dinv_hbm,
             y_v, acc_v, srcb0, srcb1, dstb0, dstb1, idx_v, deg2d, asl2d,
             dinv_sl, y_sl, h_sl, par_v, y_sh, acc_sh, sem_e, sem_r):
  sid = lax.axis_index("s")
  nb = sid * NSL
  ebase = sid * EPT
  iota = lax.iota(jnp.int32, L)
  ones16 = jnp.ones((L,), jnp.float32)
  zero16 = jnp.zeros((L,), jnp.float32)

  pltpu.sync_copy(idx_hbm, idx_v)
  pltpu.sync_copy(par_hbm, par_v)

  sbufs = (srcb0, srcb1)
  dbufs = (dstb0, dstb1)

  def _reduce(nch):
    descs = [pltpu.async_copy(acc_v.at[pl.ds(c * RCH, RCH)],
                              acc_sh.at[idx_v.at[c]], sem_r, add=True)
             for c in range(nch)]
    for de in descs:
      de.wait()

  d0 = pltpu.async_copy(dst_hbm.at[pl.ds(ebase, EPC)], dbufs[0], sem_e)

  @plsc.parallel_loop(0, ROWS1 * 8 // L, unroll=4)
  def _z0(i):
    w = i * L + iota
    plsc.store_scatter(acc_v, [w >> 3, w & 7], zero16)

  @pl.when(sid == 0)
  def _():
    pltpu.sync_copy(acc_v.at[pl.ds(0, ROWS1)], acc_sh.at[pl.ds(0, ROWS1)])

  d0.wait()
  plsc.subcore_barrier()

  for q in range(NQ):
    bi = q & 1
    if q < NQ - 1:
      dn = pltpu.async_copy(
          dst_hbm.at[pl.ds(ebase + (q + 1) * EPC, EPC)], dbufs[1 - bi], sem_e)
    dcur = dbufs[bi]

    @plsc.parallel_loop(0, EPC // L, unroll=4)
    def _deg(i):
      d = dcur[pl.ds(i * L, L)]
      plsc.addupdate_scatter(acc_v, [d >> 3, d & 7], ones16)

    if q < NQ - 1:
      dn.wait()

  _reduce(ROWS1 // RCH)
  plsc.subcore_barrier()

  pltpu.sync_copy(acc_sh.at[pl.ds(sid * (NSL // 8), NSL // 8)], deg2d)

  @plsc.parallel_loop(0, NSL // L, unroll=2)
  def _dv(i):
    w = i * L + iota
    dg = plsc.load_gather(deg2d, [w >> 3, w & 7]) + 1.0
    ib = plsc.bitcast(dg, jnp.int32)
    ib = jnp.int32(0x5F3759DF) - (ib >> 1)
    yv = plsc.bitcast(ib, jnp.float32)
    for _ in range(4):
      yv = yv * (1.5 - 0.5 * dg * yv * yv)
    dinv_sl[pl.ds(i * L, L)] = yv

  pltpu.sync_copy(dinv_sl, dinv_hbm.at[pl.ds(nb, NSL)])

  pltpu.sync_copy(xw1_hbm.at[pl.ds(nb * 4, NSL * 4)], y_sl)

  @plsc.parallel_loop(0, NSL * 4 // L, unroll=2)
  def _y1(i):
    w = i * L + iota
    dv = plsc.load_gather(dinv_sl, [w >> 2])
    y_sl[pl.ds(i * L, L)] = y_sl[pl.ds(i * L, L)] * dv

  pltpu.sync_copy(y_sl, y_sh.at[pl.ds(sid * (NSL * 4), NSL * 4)])
  plsc.subcore_barrier()

  for li, F in enumerate((4, 4, 2)):
    rows = NPAD * F // 8

    yp = pltpu.async_copy(y_sh.at[pl.ds(0, NPAD * F)],
                          y_v.at[pl.ds(0, NPAD * F)], sem_r)
    s0 = pltpu.async_copy(src_hbm.at[pl.ds(ebase, EPC)], sbufs[0], sem_e)
    e0 = pltpu.async_copy(dst_hbm.at[pl.ds(ebase, EPC)], dbufs[0], sem_e)

    @plsc.parallel_loop(0, rows * 8 // L, unroll=4)
    def _z(i):
      w = i * L + iota
      plsc.store_scatter(acc_v, [w >> 3, w & 7], zero16)

    @pl.when(sid == 0)
    def _():
      pltpu.sync_copy(acc_v.at[pl.ds(0, rows)], acc_sh.at[pl.ds(0, rows)])

    yp.wait()
    s0.wait()
    e0.wait()
    plsc.subcore_barrier()

    for q in range(NQ):
      bi = q & 1
      if q < NQ - 1:
        sn = pltpu.async_copy(
            src_hbm.at[pl.ds(ebase + (q + 1) * EPC, EPC)], sbufs[1 - bi], sem_e)
        en = pltpu.async_copy(
            dst_hbm.at[pl.ds(ebase + (q + 1) * EPC, EPC)], dbufs[1 - bi], sem_e)
      scur = sbufs[bi]
      dcur = dbufs[bi]

      @plsc.parallel_loop(0, EPC // L, unroll=4)
      def _e(i):
        s = scur[pl.ds(i * L, L)] * F
        d = dcur[pl.ds(i * L, L)] * F
        for f in range(F):
          v = plsc.load_gather(y_v, [s + f])
          w = d + f
          plsc.addupdate_scatter(acc_v, [w >> 3, w & 7], v)

      if q < NQ - 1:
        sn.wait()
        en.wait()

    _reduce(rows // RCH)
    plsc.subcore_barrier()

    if li < 2:
      F_out = 4 if li == 0 else 2
      woff = 0 if li == 0 else 16
      boff = 24 if li == 0 else 28
      srow = NSL * F // 8
      pltpu.sync_copy(acc_sh.at[pl.ds(sid * srow, srow)],
                      asl2d.at[pl.ds(0, srow)])

      @plsc.parallel_loop(0, NSL * F // L, unroll=2)
      def _h(i):
        w = i * L + iota
        a = plsc.load_gather(asl2d, [w >> 3, w & 7])
        yv = y_sl[pl.ds(i * L, L)]
        dv = plsc.load_gather(dinv_sl, [w >> 2])
        b = plsc.load_gather(par_v, [(w & 3) + boff])
        xx = dv * (a + yv) + b
        ex = jnp.exp(2.0 * xx)
        h_sl[pl.ds(i * L, L)] = 1.0 - 2.0 / (ex + 1.0)

      sh_out = 2 if F_out == 4 else 1

      @plsc.parallel_loop(0, NSL * F_out // L, unroll=2)
      def _y(i):
        w2 = i * L + iota
        n = w2 >> sh_out
        g = w2 & (F_out - 1)
        s = zero16
        for f in range(4):
          hv = plsc.load_gather(h_sl, [n * 4 + f])
          wv = plsc.load_gather(par_v, [woff + f * F_out + g])
          s = s + hv * wv
        dv = plsc.load_gather(dinv_sl, [n])
        y_sl[pl.ds(i * L, L)] = dv * s

      slw = NSL * F_out
      pltpu.sync_copy(y_sl.at[pl.ds(0, slw)],
                      y_sh.at[pl.ds(sid * slw, slw)])
      plsc.subcore_barrier()
    else:
      myr = rows // NS
      pltpu.sync_copy(acc_sh.at[pl.ds(sid * myr, myr)],
                      agg3_hbm.at[pl.ds(sid * myr, myr)])
      pltpu.sync_copy(y_sl.at[pl.ds(0, NSL * 2)],
                      y3_hbm.at[pl.ds(sid * (NSL * 2), NSL * 2)])


def _xw1_body(x_ref, w1_ref, o_ref):
  o_ref[:N, :] = jnp.dot(x_ref[...], w1_ref[...],
                         preferred_element_type=jnp.float32)
  o_ref[N:, :] = jnp.zeros((NPAD - N, 4), jnp.float32)


def _final_body(agg_ref, y_ref, dinv_ref, b3_ref, wc_ref, bc_ref,
                out_ref, h_ref):
  a = agg_ref[:N, :] + y_ref[:N, :]
  h3 = jnp.tanh(dinv_ref[...] * a + b3_ref[...])
  logits = jnp.dot(h3, wc_ref[...], preferred_element_type=jnp.float32) + bc_ref[...]
  m = jnp.max(logits, axis=1, keepdims=True)
  e = logits - m
  lse = jnp.log(jnp.sum(jnp.exp(e), axis=1, keepdims=True))
  out_ref[...] = e - lse
  h_ref[...] = h3


_xw1 = pl.pallas_call(
    _xw1_body, out_shape=jax.ShapeDtypeStruct((NPAD, 4), jnp.float32))
_final = pl.pallas_call(
    _final_body,
    out_shape=(jax.ShapeDtypeStruct((N, 4), jnp.float32),
               jax.ShapeDtypeStruct((N, 2), jnp.float32)))


@jax.jit
def kernel(x, edge_index, W1, b1, W2, b2, W3, b3, Wc, bc):
  src = edge_index[0].astype(jnp.int32)
  dst = edge_index[1].astype(jnp.int32)
  idx = jnp.arange(ROWS4, dtype=jnp.int32).reshape(ROWS4 // RCH, RCH)
  par = jnp.concatenate([W2.reshape(-1), W3.reshape(-1),
                         b1.reshape(-1), b2.reshape(-1)]).astype(jnp.float32)

  xw1 = _xw1(x, W1).reshape(-1)
  agg3, y3, dinv = _sc_mega(xw1, src, dst, idx, par)

  out, h = _final(agg3.reshape(NPAD, 2), y3.reshape(NPAD, 2),
                  dinv[:N].reshape(N, 1), b3.reshape(1, 2), Wc,
                  bc.reshape(1, 4))
  return out, h


# speedup vs baseline: 88.6802x; 1.2092x over previous
import functools

import jax
import jax.numpy as jnp
from jax import lax
from jax.experimental import pallas as pl
from jax.experimental.pallas import tpu as pltpu
from jax.experimental.pallas import tpu_sc as plsc

N = 10000
E = 320000
NS = 16
L = 16
EPT = E // NS
NQ = 5
EPC = EPT // NQ
NPAD = 10240
NSL = NPAD // NS
RCH = 128
ROWS4 = NPAD * 4 // 8
ROWS2 = NPAD * 2 // 8
ROWS1 = NPAD * 1 // 8

_mesh = plsc.VectorSubcoreMesh(core_axis_name="c", subcore_axis_name="s",
                               num_cores=1)


def _rcp(d):
  r = 1.0 / d
  return r * (2.0 - d * r)


@functools.partial(
    pl.kernel,
    out_type=(jax.ShapeDtypeStruct((NPAD, 4), jnp.float32),
              jax.ShapeDtypeStruct((NPAD, 2), jnp.float32)),
    mesh=_mesh,
    compiler_params=pltpu.CompilerParams(needs_layout_passes=False,
                                         use_tc_tiling_on_sc=False),
    scratch_types=[
        pltpu.VMEM((NPAD * 4,), jnp.float32),
        pltpu.VMEM((ROWS4, 8), jnp.float32),
        pltpu.VMEM((EPC,), jnp.int32),
        pltpu.VMEM((EPC,), jnp.int32),
        pltpu.VMEM((EPC,), jnp.int32),
        pltpu.VMEM((EPC,), jnp.int32),
        pltpu.VMEM((ROWS4 // RCH, RCH), jnp.int32),
        pltpu.VMEM((NSL // 8, 8), jnp.float32),
        pltpu.VMEM((NSL * 4 // 8, 8), jnp.float32),
        pltpu.VMEM((NSL,), jnp.float32),
        pltpu.VMEM((NSL * 4,), jnp.float32),
        pltpu.VMEM((NSL * 4,), jnp.float32),
        pltpu.VMEM((NSL, 2), jnp.float32),
        pltpu.VMEM((NSL, 4), jnp.float32),
        pltpu.VMEM((48,), jnp.float32),
        pltpu.VMEM_SHARED((NPAD * 4,), jnp.float32),
        pltpu.VMEM_SHARED((ROWS4, 8), jnp.float32),
        pltpu.SemaphoreType.DMA,
        pltpu.SemaphoreType.DMA,
    ],
)
def _sc_mega(xw1_hbm, ei_hbm, idx_hbm, par_hbm,
             out_hbm, h_hbm,
             y_v, acc_v, srcb0, srcb1, dstb0, dstb1, idx_v, deg2d, asl2d,
             dinv_sl, y_sl, h_sl, hb2d, ob2d, par_v, y_sh, acc_sh,
             sem_e, sem_r):
  sid = lax.axis_index("s")
  nb = sid * NSL
  ebase = sid * EPT
  iota = lax.iota(jnp.int32, L)
  ones16 = jnp.ones((L,), jnp.float32)
  zero16 = jnp.zeros((L,), jnp.float32)

  pltpu.sync_copy(idx_hbm, idx_v)
  pltpu.sync_copy(par_hbm, par_v)

  sbufs = (srcb0, srcb1)
  dbufs = (dstb0, dstb1)

  def _reduce(nch):
    descs = [pltpu.async_copy(acc_v.at[pl.ds(c * RCH, RCH)],
                              acc_sh.at[idx_v.at[c]], sem_r, add=True)
             for c in range(nch)]
    for de in descs:
      de.wait()

  d0 = pltpu.async_copy(ei_hbm.at[1, pl.ds(ebase, EPC)], dbufs[0], sem_e)

  @plsc.parallel_loop(0, ROWS1 * 8 // L, unroll=4)
  def _z0(i):
    w = i * L + iota
    plsc.store_scatter(acc_v, [w >> 3, w & 7], zero16)

  @pl.when(sid == 0)
  def _():
    pltpu.sync_copy(acc_v.at[pl.ds(0, ROWS1)], acc_sh.at[pl.ds(0, ROWS1)])

  d0.wait()
  plsc.subcore_barrier()

  for q in range(NQ):
    bi = q & 1
    if q < NQ - 1:
      dn = pltpu.async_copy(
          ei_hbm.at[1, pl.ds(ebase + (q + 1) * EPC, EPC)], dbufs[1 - bi],
          sem_e)
    dcur = dbufs[bi]

    @plsc.parallel_loop(0, EPC // L, unroll=4)
    def _deg(i):
      d = dcur[pl.ds(i * L, L)]
      plsc.addupdate_scatter(acc_v, [d >> 3, d & 7], ones16)

    if q < NQ - 1:
      dn.wait()

  _reduce(ROWS1 // RCH)
  plsc.subcore_barrier()

  pltpu.sync_copy(acc_sh.at[pl.ds(sid * (NSL // 8), NSL // 8)], deg2d)

  @plsc.parallel_loop(0, NSL // L, unroll=2)
  def _dv(i):
    w = i * L + iota
    dg = plsc.load_gather(deg2d, [w >> 3, w & 7]) + 1.0
    ib = plsc.bitcast(dg, jnp.int32)
    ib = jnp.int32(0x5F3759DF) - (ib >> 1)
    yv = plsc.bitcast(ib, jnp.float32)
    for _ in range(4):
      yv = yv * (1.5 - 0.5 * dg * yv * yv)
    dinv_sl[pl.ds(i * L, L)] = yv

  pltpu.sync_copy(xw1_hbm.at[pl.ds(nb, NSL)], ob2d)

  @plsc.parallel_loop(0, NSL * 4 // L, unroll=2)
  def _y1(i):
    w = i * L + iota
    xv = plsc.load_gather(ob2d, [w >> 2, w & 3])
    dv = plsc.load_gather(dinv_sl, [w >> 2])
    y_sl[pl.ds(i * L, L)] = xv * dv

  pltpu.sync_copy(y_sl, y_sh.at[pl.ds(sid * (NSL * 4), NSL * 4)])
  plsc.subcore_barrier()

  for li, F in enumerate((4, 4, 2)):
    rows = NPAD * F // 8

    yp = pltpu.async_copy(y_sh.at[pl.ds(0, NPAD * F)],
                          y_v.at[pl.ds(0, NPAD * F)], sem_r)
    s0 = pltpu.async_copy(ei_hbm.at[0, pl.ds(ebase, EPC)], sbufs[0], sem_e)
    e0 = pltpu.async_copy(ei_hbm.at[1, pl.ds(ebase, EPC)], dbufs[0], sem_e)

    @plsc.parallel_loop(0, rows * 8 // L, unroll=4)
    def _z(i):
      w = i * L + iota
      plsc.store_scatter(acc_v, [w >> 3, w & 7], zero16)

    @pl.when(sid == 0)
    def _():
      pltpu.sync_copy(acc_v.at[pl.ds(0, rows)], acc_sh.at[pl.ds(0, rows)])

    yp.wait()
    s0.wait()
    e0.wait()
    plsc.subcore_barrier()

    for q in range(NQ):
      bi = q & 1
      if q < NQ - 1:
        sn = pltpu.async_copy(
            ei_hbm.at[0, pl.ds(ebase + (q + 1) * EPC, EPC)], sbufs[1 - bi],
            sem_e)
        en = pltpu.async_copy(
            ei_hbm.at[1, pl.ds(ebase + (q + 1) * EPC, EPC)], dbufs[1 - bi],
            sem_e)
      scur = sbufs[bi]
      dcur = dbufs[bi]

      @plsc.parallel_loop(0, EPC // L, unroll=4)
      def _e(i):
        s = scur[pl.ds(i * L, L)] * F
        d = dcur[pl.ds(i * L, L)] * F
        for f in range(F):
          v = plsc.load_gather(y_v, [s + f])
          w = d + f
          plsc.addupdate_scatter(acc_v, [w >> 3, w & 7], v)

      if q < NQ - 1:
        sn.wait()
        en.wait()

    _reduce(rows // RCH)
    plsc.subcore_barrier()

    if li < 2:
      F_out = 4 if li == 0 else 2
      woff = 0 if li == 0 else 16
      boff = 24 if li == 0 else 28
      srow = NSL * F // 8
      pltpu.sync_copy(acc_sh.at[pl.ds(sid * srow, srow)],
                      asl2d.at[pl.ds(0, srow)])

      @plsc.parallel_loop(0, NSL * F // L, unroll=2)
      def _h(i):
        w = i * L + iota
        a = plsc.load_gather(asl2d, [w >> 3, w & 7])
        yv = y_sl[pl.ds(i * L, L)]
        dv = plsc.load_gather(dinv_sl, [w >> 2])
        b = plsc.load_gather(par_v, [(w & 3) + boff])
        xx = dv * (a + yv) + b
        ex = jnp.exp(2.0 * xx)
        h_sl[pl.ds(i * L, L)] = 1.0 - 2.0 * _rcp(ex + 1.0)

      sh_out = 2 if F_out == 4 else 1

      @plsc.parallel_loop(0, NSL * F_out // L, unroll=2)
      def _y(i):
        w2 = i * L + iota
        n = w2 >> sh_out
        g = w2 & (F_out - 1)
        s = zero16
        for f in range(4):
          hv = plsc.load_gather(h_sl, [n * 4 + f])
          wv = plsc.load_gather(par_v, [woff + f * F_out + g])
          s = s + hv * wv
        dv = plsc.load_gather(dinv_sl, [n])
        y_sl[pl.ds(i * L, L)] = dv * s

      slw = NSL * F_out
      pltpu.sync_copy(y_sl.at[pl.ds(0, slw)],
                      y_sh.at[pl.ds(sid * slw, slw)])
      plsc.subcore_barrier()
    else:
      srow2 = NSL * 2 // 8
      pltpu.sync_copy(acc_sh.at[pl.ds(sid * srow2, srow2)],
                      asl2d.at[pl.ds(0, srow2)])

      @plsc.parallel_loop(0, NSL * 2 // L, unroll=2)
      def _h3(i):
        w = i * L + iota
        a = plsc.load_gather(asl2d, [w >> 3, w & 7])
        yv = y_sl[pl.ds(i * L, L)]
        dv = plsc.load_gather(dinv_sl, [w >> 1])
        b = plsc.load_gather(par_v, [(w & 1) + 32])
        xx = dv * (a + yv) + b
        ex = jnp.exp(2.0 * xx)
        h3 = 1.0 - 2.0 * _rcp(ex + 1.0)
        plsc.store_scatter(hb2d, [w >> 1, w & 1], h3)

      pltpu.sync_copy(hb2d, h_hbm.at[pl.ds(nb, NSL)])

      @plsc.parallel_loop(0, NSL * 4 // L, unroll=2)
      def _lg(i):
        w4 = i * L + iota
        n = w4 >> 2
        j = w4 & 3
        s = plsc.load_gather(par_v, [j + 42])
        for k in range(2):
          hv = plsc.load_gather(hb2d, [n, jnp.full((L,), k, jnp.int32)])
          wv = plsc.load_gather(par_v, [34 + k * 4 + j])
          s = s + hv * wv
        y_sl[pl.ds(i * L, L)] = s

      @plsc.parallel_loop(0, NSL // L, unroll=2)
      def _ls(i):
        n = i * L + iota
        l0 = plsc.load_gather(y_sl, [n * 4])
        l1 = plsc.load_gather(y_sl, [n * 4 + 1])
        l2 = plsc.load_gather(y_sl, [n * 4 + 2])
        l3 = plsc.load_gather(y_sl, [n * 4 + 3])
        m = jnp.maximum(jnp.maximum(l0, l1), jnp.maximum(l2, l3))
        s = (jnp.exp(l0 - m) + jnp.exp(l1 - m) +
             jnp.exp(l2 - m) + jnp.exp(l3 - m))
        ibits = plsc.bitcast(s, jnp.int32)
        ev = ((ibits >> 23) & 0xFF) - 127
        mant = plsc.bitcast((ibits & 0x7FFFFF) | 0x3F800000, jnp.float32)
        t = (mant - 1.0) * _rcp(mant + 1.0)
        t2 = t * t
        lnm = 2.0 * t * (1.0 + t2 * (1.0 / 3.0 + t2 * (0.2 + t2 * (
            1.0 / 7.0 + t2 * (1.0 / 9.0)))))
        lse = m + ev.astype(jnp.float32) * 0.6931471805599453 + lnm
        plsc.store_scatter(deg2d, [n >> 3, n & 7], lse)

      @plsc.parallel_loop(0, NSL * 4 // L, unroll=2)
      def _out(i):
        w4 = i * L + iota
        l = y_sl[pl.ds(i * L, L)]
        n4 = w4 >> 2
        ls = plsc.load_gather(deg2d, [n4 >> 3, n4 & 7])
        plsc.store_scatter(ob2d, [n4, w4 & 3], l - ls)

      pltpu.sync_copy(ob2d, out_hbm.at[pl.ds(nb, NSL)])


def _xw1_body(x_ref, w1_ref, o_ref):
  o_ref[:N, :] = jnp.dot(x_ref[...], w1_ref[...],
                         preferred_element_type=jnp.float32)
  o_ref[N:, :] = jnp.zeros((NPAD - N, 4), jnp.float32)


_xw1 = pl.pallas_call(
    _xw1_body, out_shape=jax.ShapeDtypeStruct((NPAD, 4), jnp.float32))


@jax.jit
def kernel(x, edge_index, W1, b1, W2, b2, W3, b3, Wc, bc):
  ei = edge_index.astype(jnp.int32)
  idx = jnp.arange(ROWS4, dtype=jnp.int32).reshape(ROWS4 // RCH, RCH)
  par = jnp.concatenate([
      W2.reshape(-1), W3.reshape(-1), b1.reshape(-1), b2.reshape(-1),
      b3.reshape(-1), Wc.reshape(-1), bc.reshape(-1),
      jnp.zeros((2,), jnp.float32)]).astype(jnp.float32)

  xw1 = _xw1(x, W1)
  out_full, h_full = _sc_mega(xw1, ei, idx, par)
  return out_full[:N], h_full[:N]


# speedup vs baseline: 90.2117x; 1.0173x over previous
import functools

import jax
import jax.numpy as jnp
from jax import lax
from jax.experimental import pallas as pl
from jax.experimental.pallas import tpu as pltpu
from jax.experimental.pallas import tpu_sc as plsc

N = 10000
E = 320000
NS = 16
L = 16
EPT = E // NS
NQ = 5
EPC = EPT // NQ
NPAD = 10240
NSL = NPAD // NS
RCH = 128
ROWS4 = NPAD * 4 // 8
ROWS2 = NPAD * 2 // 8
ROWS1 = NPAD * 1 // 8

_mesh = plsc.VectorSubcoreMesh(core_axis_name="c", subcore_axis_name="s",
                               num_cores=1)


def _rcp(d):
  r = 1.0 / d
  return r * (2.0 - d * r)


@functools.partial(
    pl.kernel,
    out_type=(jax.ShapeDtypeStruct((N, 4), jnp.float32),
              jax.ShapeDtypeStruct((N, 2), jnp.float32)),
    mesh=_mesh,
    compiler_params=pltpu.CompilerParams(needs_layout_passes=False,
                                         use_tc_tiling_on_sc=False),
    scratch_types=[
        pltpu.VMEM((NPAD * 4,), jnp.float32),
        pltpu.VMEM((ROWS4, 8), jnp.float32),
        pltpu.VMEM((EPC,), jnp.int32),
        pltpu.VMEM((EPC,), jnp.int32),
        pltpu.VMEM((EPC,), jnp.int32),
        pltpu.VMEM((EPC,), jnp.int32),
        pltpu.VMEM((ROWS4 // RCH, RCH), jnp.int32),
        pltpu.VMEM((NSL // 8, 8), jnp.float32),
        pltpu.VMEM((NSL * 4 // 8, 8), jnp.float32),
        pltpu.VMEM((NSL,), jnp.float32),
        pltpu.VMEM((NSL * 4,), jnp.float32),
        pltpu.VMEM((NSL * 4,), jnp.float32),
        pltpu.VMEM((NSL, 2), jnp.float32),
        pltpu.VMEM((NSL, 4), jnp.float32),
        pltpu.VMEM((48,), jnp.float32),
        pltpu.VMEM_SHARED((NPAD * 4,), jnp.float32),
        pltpu.VMEM_SHARED((ROWS4, 8), jnp.float32),
        pltpu.SemaphoreType.DMA,
        pltpu.SemaphoreType.DMA,
    ],
)
def _sc_mega(xw1_hbm, ei_hbm, idx_hbm, par_hbm,
             out_hbm, h_hbm,
             y_v, acc_v, srcb0, srcb1, dstb0, dstb1, idx_v, deg2d, asl2d,
             dinv_sl, y_sl, h_sl, hb2d, ob2d, par_v, y_sh, acc_sh,
             sem_e, sem_r):
  sid = lax.axis_index("s")
  nb = sid * NSL
  ebase = sid * EPT
  iota = lax.iota(jnp.int32, L)
  ones16 = jnp.ones((L,), jnp.float32)
  zero16 = jnp.zeros((L,), jnp.float32)

  pltpu.sync_copy(idx_hbm, idx_v)
  pltpu.sync_copy(par_hbm, par_v)

  sbufs = (srcb0, srcb1)
  dbufs = (dstb0, dstb1)

  def _reduce(nch):
    descs = [pltpu.async_copy(acc_v.at[pl.ds(c * RCH, RCH)],
                              acc_sh.at[idx_v.at[c]], sem_r, add=True)
             for c in range(nch)]
    for de in descs:
      de.wait()

  d0 = pltpu.async_copy(ei_hbm.at[1, pl.ds(ebase, EPC)], dbufs[0], sem_e)

  @plsc.parallel_loop(0, ROWS1 * 8 // L, unroll=4)
  def _z0(i):
    w = i * L + iota
    plsc.store_scatter(acc_v, [w >> 3, w & 7], zero16)

  @pl.when(sid == 0)
  def _():
    pltpu.sync_copy(acc_v.at[pl.ds(0, ROWS1)], acc_sh.at[pl.ds(0, ROWS1)])

  d0.wait()
  plsc.subcore_barrier()

  for q in range(NQ):
    bi = q & 1
    if q < NQ - 1:
      dn = pltpu.async_copy(
          ei_hbm.at[1, pl.ds(ebase + (q + 1) * EPC, EPC)], dbufs[1 - bi],
          sem_e)
    dcur = dbufs[bi]

    @plsc.parallel_loop(0, EPC // L, unroll=8)
    def _deg(i):
      d = dcur[pl.ds(i * L, L)]
      plsc.addupdate_scatter(acc_v, [d >> 3, d & 7], ones16)

    if q < NQ - 1:
      dn.wait()

  _reduce(ROWS1 // RCH)
  plsc.subcore_barrier()

  pltpu.sync_copy(acc_sh.at[pl.ds(sid * (NSL // 8), NSL // 8)], deg2d)

  @plsc.parallel_loop(0, NSL // L, unroll=2)
  def _dv(i):
    w = i * L + iota
    dg = plsc.load_gather(deg2d, [w >> 3, w & 7]) + 1.0
    ib = plsc.bitcast(dg, jnp.int32)
    ib = jnp.int32(0x5F3759DF) - (ib >> 1)
    yv = plsc.bitcast(ib, jnp.float32)
    for _ in range(4):
      yv = yv * (1.5 - 0.5 * dg * yv * yv)
    dinv_sl[pl.ds(i * L, L)] = yv

  pltpu.sync_copy(xw1_hbm.at[pl.ds(nb, NSL)], ob2d)

  @plsc.parallel_loop(0, NSL * 4 // L, unroll=2)
  def _y1(i):
    w = i * L + iota
    xv = plsc.load_gather(ob2d, [w >> 2, w & 3])
    dv = plsc.load_gather(dinv_sl, [w >> 2])
    y_sl[pl.ds(i * L, L)] = xv * dv

  pltpu.sync_copy(y_sl, y_sh.at[pl.ds(sid * (NSL * 4), NSL * 4)])
  plsc.subcore_barrier()

  for li, F in enumerate((4, 4, 2)):
    rows = NPAD * F // 8

    yp = pltpu.async_copy(y_sh.at[pl.ds(0, NPAD * F)],
                          y_v.at[pl.ds(0, NPAD * F)], sem_r)
    s0 = pltpu.async_copy(ei_hbm.at[0, pl.ds(ebase, EPC)], sbufs[0], sem_e)
    e0 = pltpu.async_copy(ei_hbm.at[1, pl.ds(ebase, EPC)], dbufs[0], sem_e)

    @plsc.parallel_loop(0, rows * 8 // L, unroll=4)
    def _z(i):
      w = i * L + iota
      plsc.store_scatter(acc_v, [w >> 3, w & 7], zero16)

    @pl.when(sid == 0)
    def _():
      pltpu.sync_copy(acc_v.at[pl.ds(0, rows)], acc_sh.at[pl.ds(0, rows)])

    yp.wait()
    s0.wait()
    e0.wait()
    plsc.subcore_barrier()

    for q in range(NQ):
      bi = q & 1
      if q < NQ - 1:
        sn = pltpu.async_copy(
            ei_hbm.at[0, pl.ds(ebase + (q + 1) * EPC, EPC)], sbufs[1 - bi],
            sem_e)
        en = pltpu.async_copy(
            ei_hbm.at[1, pl.ds(ebase + (q + 1) * EPC, EPC)], dbufs[1 - bi],
            sem_e)
      scur = sbufs[bi]
      dcur = dbufs[bi]

      @plsc.parallel_loop(0, EPC // L, unroll=8)
      def _e(i):
        s = scur[pl.ds(i * L, L)] * F
        d = dcur[pl.ds(i * L, L)] * F
        for f in range(F):
          v = plsc.load_gather(y_v, [s + f])
          w = d + f
          plsc.addupdate_scatter(acc_v, [w >> 3, w & 7], v)

      if q < NQ - 1:
        sn.wait()
        en.wait()

    _reduce(rows // RCH)
    plsc.subcore_barrier()

    if li < 2:
      F_out = 4 if li == 0 else 2
      woff = 0 if li == 0 else 16
      boff = 24 if li == 0 else 28
      srow = NSL * F // 8
      pltpu.sync_copy(acc_sh.at[pl.ds(sid * srow, srow)],
                      asl2d.at[pl.ds(0, srow)])

      @plsc.parallel_loop(0, NSL * F // L, unroll=2)
      def _h(i):
        w = i * L + iota
        a = plsc.load_gather(asl2d, [w >> 3, w & 7])
        yv = y_sl[pl.ds(i * L, L)]
        dv = plsc.load_gather(dinv_sl, [w >> 2])
        b = plsc.load_gather(par_v, [(w & 3) + boff])
        xx = dv * (a + yv) + b
        ex = jnp.exp(2.0 * xx)
        h_sl[pl.ds(i * L, L)] = 1.0 - 2.0 * _rcp(ex + 1.0)

      sh_out = 2 if F_out == 4 else 1

      @plsc.parallel_loop(0, NSL * F_out // L, unroll=2)
      def _y(i):
        w2 = i * L + iota
        n = w2 >> sh_out
        g = w2 & (F_out - 1)
        s = zero16
        for f in range(4):
          hv = plsc.load_gather(h_sl, [n * 4 + f])
          wv = plsc.load_gather(par_v, [woff + f * F_out + g])
          s = s + hv * wv
        dv = plsc.load_gather(dinv_sl, [n])
        y_sl[pl.ds(i * L, L)] = dv * s

      slw = NSL * F_out
      pltpu.sync_copy(y_sl.at[pl.ds(0, slw)],
                      y_sh.at[pl.ds(sid * slw, slw)])
      plsc.subcore_barrier()
    else:
      srow2 = NSL * 2 // 8
      pltpu.sync_copy(acc_sh.at[pl.ds(sid * srow2, srow2)],
                      asl2d.at[pl.ds(0, srow2)])

      @plsc.parallel_loop(0, NSL * 2 // L, unroll=2)
      def _h3(i):
        w = i * L + iota
        a = plsc.load_gather(asl2d, [w >> 3, w & 7])
        yv = y_sl[pl.ds(i * L, L)]
        dv = plsc.load_gather(dinv_sl, [w >> 1])
        b = plsc.load_gather(par_v, [(w & 1) + 32])
        xx = dv * (a + yv) + b
        ex = jnp.exp(2.0 * xx)
        h3 = 1.0 - 2.0 * _rcp(ex + 1.0)
        plsc.store_scatter(hb2d, [w >> 1, w & 1], h3)

      @pl.when(sid < NS - 1)
      def _():
        pltpu.sync_copy(hb2d, h_hbm.at[pl.ds(nb, NSL)])

      @pl.when(sid == NS - 1)
      def _():
        pltpu.sync_copy(hb2d.at[pl.ds(0, N - (NS - 1) * NSL)],
                        h_hbm.at[pl.ds(nb, N - (NS - 1) * NSL)])

      @plsc.parallel_loop(0, NSL * 4 // L, unroll=2)
      def _lg(i):
        w4 = i * L + iota
        n = w4 >> 2
        j = w4 & 3
        s = plsc.load_gather(par_v, [j + 42])
        for k in range(2):
          hv = plsc.load_gather(hb2d, [n, jnp.full((L,), k, jnp.int32)])
          wv = plsc.load_gather(par_v, [34 + k * 4 + j])
          s = s + hv * wv
        y_sl[pl.ds(i * L, L)] = s

      @plsc.parallel_loop(0, NSL // L, unroll=2)
      def _ls(i):
        n = i * L + iota
        l0 = plsc.load_gather(y_sl, [n * 4])
        l1 = plsc.load_gather(y_sl, [n * 4 + 1])
        l2 = plsc.load_gather(y_sl, [n * 4 + 2])
        l3 = plsc.load_gather(y_sl, [n * 4 + 3])
        m = jnp.maximum(jnp.maximum(l0, l1), jnp.maximum(l2, l3))
        s = (jnp.exp(l0 - m) + jnp.exp(l1 - m) +
             jnp.exp(l2 - m) + jnp.exp(l3 - m))
        ibits = plsc.bitcast(s, jnp.int32)
        ev = ((ibits >> 23) & 0xFF) - 127
        mant = plsc.bitcast((ibits & 0x7FFFFF) | 0x3F800000, jnp.float32)
        t = (mant - 1.0) * _rcp(mant + 1.0)
        t2 = t * t
        lnm = 2.0 * t * (1.0 + t2 * (1.0 / 3.0 + t2 * (0.2 + t2 * (
            1.0 / 7.0 + t2 * (1.0 / 9.0)))))
        lse = m + ev.astype(jnp.float32) * 0.6931471805599453 + lnm
        plsc.store_scatter(deg2d, [n >> 3, n & 7], lse)

      @plsc.parallel_loop(0, NSL * 4 // L, unroll=2)
      def _out(i):
        w4 = i * L + iota
        l = y_sl[pl.ds(i * L, L)]
        n4 = w4 >> 2
        ls = plsc.load_gather(deg2d, [n4 >> 3, n4 & 7])
        plsc.store_scatter(ob2d, [n4, w4 & 3], l - ls)

      @pl.when(sid < NS - 1)
      def _():
        pltpu.sync_copy(ob2d, out_hbm.at[pl.ds(nb, NSL)])

      @pl.when(sid == NS - 1)
      def _():
        pltpu.sync_copy(ob2d.at[pl.ds(0, N - (NS - 1) * NSL)],
                        out_hbm.at[pl.ds(nb, N - (NS - 1) * NSL)])


def _xw1_body(x_ref, w1_ref, o_ref):
  o_ref[:N, :] = jnp.dot(x_ref[...], w1_ref[...],
                         preferred_element_type=jnp.float32)
  o_ref[N:, :] = jnp.zeros((NPAD - N, 4), jnp.float32)


_xw1 = pl.pallas_call(
    _xw1_body, out_shape=jax.ShapeDtypeStruct((NPAD, 4), jnp.float32))


@jax.jit
def kernel(x, edge_index, W1, b1, W2, b2, W3, b3, Wc, bc):
  ei = edge_index.astype(jnp.int32)
  idx = jnp.arange(ROWS4, dtype=jnp.int32).reshape(ROWS4 // RCH, RCH)
  par = jnp.concatenate([
      W2.reshape(-1), W3.reshape(-1), b1.reshape(-1), b2.reshape(-1),
      b3.reshape(-1), Wc.reshape(-1), bc.reshape(-1),
      jnp.zeros((2,), jnp.float32)]).astype(jnp.float32)

  xw1 = _xw1(x, W1)
  out, h = _sc_mega(xw1, ei, idx, par)
  return out, h


# speedup vs baseline: 93.5185x; 1.0367x over previous
import functools

import jax
import jax.numpy as jnp
from jax import lax
from jax.experimental import pallas as pl
from jax.experimental.pallas import tpu as pltpu
from jax.experimental.pallas import tpu_sc as plsc

N = 10000
E = 320000
NS = 16
L = 16
EPT = E // NS
NQ = 5
EPC = EPT // NQ
NPAD = 10240
NSL = NPAD // NS
RCH = 128
ROWS4 = NPAD * 4 // 8
ROWS2 = NPAD * 2 // 8
ROWS1 = NPAD * 1 // 8

_mesh = plsc.VectorSubcoreMesh(core_axis_name="c", subcore_axis_name="s",
                               num_cores=1)


def _rcp(d):
  r = 1.0 / d
  return r * (2.0 - d * r)


@functools.partial(
    pl.kernel,
    out_type=(jax.ShapeDtypeStruct((N, 4), jnp.float32),
              jax.ShapeDtypeStruct((N, 2), jnp.float32)),
    mesh=_mesh,
    compiler_params=pltpu.CompilerParams(needs_layout_passes=False,
                                         use_tc_tiling_on_sc=False),
    scratch_types=[
        pltpu.VMEM((NPAD * 4,), jnp.float32),
        pltpu.VMEM((ROWS4, 8), jnp.float32),
        pltpu.VMEM((EPC,), jnp.int32),
        pltpu.VMEM((EPC,), jnp.int32),
        pltpu.VMEM((EPC,), jnp.int32),
        pltpu.VMEM((EPC,), jnp.int32),
        pltpu.VMEM((ROWS4 // RCH, RCH), jnp.int32),
        pltpu.VMEM((NSL // 8, 8), jnp.float32),
        pltpu.VMEM((NSL * 4 // 8, 8), jnp.float32),
        pltpu.VMEM((NSL,), jnp.float32),
        pltpu.VMEM((NSL * 4,), jnp.float32),
        pltpu.VMEM((NSL * 4,), jnp.float32),
        pltpu.VMEM((NSL, 2), jnp.float32),
        pltpu.VMEM((NSL, 4), jnp.float32),
        pltpu.VMEM((48,), jnp.float32),
        pltpu.VMEM_SHARED((NPAD * 4,), jnp.float32),
        pltpu.VMEM_SHARED((ROWS4, 8), jnp.float32),
        pltpu.SemaphoreType.DMA,
        pltpu.SemaphoreType.DMA,
    ],
)
def _sc_mega(xw1_hbm, ei_hbm, idx_hbm, par_hbm,
             out_hbm, h_hbm,
             y_v, acc_v, srcb0, srcb1, dstb0, dstb1, idx_v, deg2d, asl2d,
             dinv_sl, y_sl, h_sl, hb2d, ob2d, par_v, y_sh, acc_sh,
             sem_e, sem_r):
  sid = lax.axis_index("s")
  nb = sid * NSL
  ebase = sid * EPT
  iota = lax.iota(jnp.int32, L)
  ones16 = jnp.ones((L,), jnp.float32)
  zero16 = jnp.zeros((L,), jnp.float32)

  pltpu.sync_copy(idx_hbm, idx_v)
  pltpu.sync_copy(par_hbm, par_v)

  sbufs = (srcb0, srcb1)
  dbufs = (dstb0, dstb1)

  def _reduce(nch):
    descs = [pltpu.async_copy(acc_v.at[pl.ds(c * RCH, RCH)],
                              acc_sh.at[idx_v.at[c]], sem_r, add=True)
             for c in range(nch)]
    for de in descs:
      de.wait()

  d0 = pltpu.async_copy(ei_hbm.at[1, pl.ds(ebase, EPC)], dbufs[0], sem_e)

  @plsc.parallel_loop(0, ROWS1 * 8 // L, unroll=4)
  def _z0(i):
    w = i * L + iota
    plsc.store_scatter(acc_v, [w >> 3, w & 7], zero16)

  @pl.when(sid == 0)
  def _():
    pltpu.sync_copy(acc_v.at[pl.ds(0, ROWS1)], acc_sh.at[pl.ds(0, ROWS1)])

  d0.wait()
  plsc.subcore_barrier()

  for q in range(NQ):
    bi = q & 1
    if q < NQ - 1:
      dn = pltpu.async_copy(
          ei_hbm.at[1, pl.ds(ebase + (q + 1) * EPC, EPC)], dbufs[1 - bi],
          sem_e)
    dcur = dbufs[bi]

    @plsc.parallel_loop(0, EPC // L, unroll=8)
    def _deg(i):
      d = dcur[pl.ds(i * L, L)]
      plsc.addupdate_scatter(acc_v, [d >> 3, d & 7], ones16)

    if q < NQ - 1:
      dn.wait()

  _reduce(ROWS1 // RCH)
  plsc.subcore_barrier()

  pltpu.sync_copy(acc_sh.at[pl.ds(sid * (NSL // 8), NSL // 8)], deg2d)

  @plsc.parallel_loop(0, NSL // L, unroll=2)
  def _dv(i):
    w = i * L + iota
    dg = plsc.load_gather(deg2d, [w >> 3, w & 7]) + 1.0
    ib = plsc.bitcast(dg, jnp.int32)
    ib = jnp.int32(0x5F3759DF) - (ib >> 1)
    yv = plsc.bitcast(ib, jnp.float32)
    for _ in range(4):
      yv = yv * (1.5 - 0.5 * dg * yv * yv)
    dinv_sl[pl.ds(i * L, L)] = yv

  pltpu.sync_copy(xw1_hbm.at[pl.ds(nb, NSL)], ob2d)

  @plsc.parallel_loop(0, NSL * 4 // L, unroll=2)
  def _y1(i):
    w = i * L + iota
    xv = plsc.load_gather(ob2d, [w >> 2, w & 3])
    dv = plsc.load_gather(dinv_sl, [w >> 2])
    y_sl[pl.ds(i * L, L)] = xv * dv

  pltpu.sync_copy(y_sl, y_sh.at[pl.ds(sid * (NSL * 4), NSL * 4)])
  plsc.subcore_barrier()

  for li, F in enumerate((4, 4, 2)):
    rows = NPAD * F // 8

    yp = pltpu.async_copy(y_sh.at[pl.ds(0, NPAD * F)],
                          y_v.at[pl.ds(0, NPAD * F)], sem_r)
    s0 = pltpu.async_copy(ei_hbm.at[0, pl.ds(ebase, EPC)], sbufs[0], sem_e)
    e0 = pltpu.async_copy(ei_hbm.at[1, pl.ds(ebase, EPC)], dbufs[0], sem_e)

    @plsc.parallel_loop(0, rows * 8 // L, unroll=4)
    def _z(i):
      w = i * L + iota
      plsc.store_scatter(acc_v, [w >> 3, w & 7], zero16)

    @pl.when(sid == 0)
    def _():
      pltpu.sync_copy(acc_v.at[pl.ds(0, rows)], acc_sh.at[pl.ds(0, rows)])

    yp.wait()
    s0.wait()
    e0.wait()
    plsc.subcore_barrier()

    for q in range(NQ):
      bi = q & 1
      if q < NQ - 1:
        sn = pltpu.async_copy(
            ei_hbm.at[0, pl.ds(ebase + (q + 1) * EPC, EPC)], sbufs[1 - bi],
            sem_e)
        en = pltpu.async_copy(
            ei_hbm.at[1, pl.ds(ebase + (q + 1) * EPC, EPC)], dbufs[1 - bi],
            sem_e)
      scur = sbufs[bi]
      dcur = dbufs[bi]

      @plsc.parallel_loop(0, EPC // L, unroll=4 if F == 4 else 8)
      def _e(i):
        s = scur[pl.ds(i * L, L)] * F
        d = dcur[pl.ds(i * L, L)] * F
        for f in range(F):
          v = plsc.load_gather(y_v, [s + f])
          w = d + f
          plsc.addupdate_scatter(acc_v, [w >> 3, w & 7], v)

      if q < NQ - 1:
        sn.wait()
        en.wait()

    _reduce(rows // RCH)
    plsc.subcore_barrier()

    if li < 2:
      F_out = 4 if li == 0 else 2
      woff = 0 if li == 0 else 16
      boff = 24 if li == 0 else 28
      srow = NSL * F // 8
      pltpu.sync_copy(acc_sh.at[pl.ds(sid * srow, srow)],
                      asl2d.at[pl.ds(0, srow)])

      @plsc.parallel_loop(0, NSL * F // L, unroll=2)
      def _h(i):
        w = i * L + iota
        a = plsc.load_gather(asl2d, [w >> 3, w & 7])
        yv = y_sl[pl.ds(i * L, L)]
        dv = plsc.load_gather(dinv_sl, [w >> 2])
        b = plsc.load_gather(par_v, [(w & 3) + boff])
        xx = dv * (a + yv) + b
        ex = jnp.exp(2.0 * xx)
        h_sl[pl.ds(i * L, L)] = 1.0 - 2.0 * _rcp(ex + 1.0)

      sh_out = 2 if F_out == 4 else 1

      @plsc.parallel_loop(0, NSL * F_out // L, unroll=2)
      def _y(i):
        w2 = i * L + iota
        n = w2 >> sh_out
        g = w2 & (F_out - 1)
        s = zero16
        for f in range(4):
          hv = plsc.load_gather(h_sl, [n * 4 + f])
          wv = plsc.load_gather(par_v, [woff + f * F_out + g])
          s = s + hv * wv
        dv = plsc.load_gather(dinv_sl, [n])
        y_sl[pl.ds(i * L, L)] = dv * s

      slw = NSL * F_out
      pltpu.sync_copy(y_sl.at[pl.ds(0, slw)],
                      y_sh.at[pl.ds(sid * slw, slw)])
      plsc.subcore_barrier()
    else:
      srow2 = NSL * 2 // 8
      pltpu.sync_copy(acc_sh.at[pl.ds(sid * srow2, srow2)],
                      asl2d.at[pl.ds(0, srow2)])

      @plsc.parallel_loop(0, NSL * 2 // L, unroll=2)
      def _h3(i):
        w = i * L + iota
        a = plsc.load_gather(asl2d, [w >> 3, w & 7])
        yv = y_sl[pl.ds(i * L, L)]
        dv = plsc.load_gather(dinv_sl, [w >> 1])
        b = plsc.load_gather(par_v, [(w & 1) + 32])
        xx = dv * (a + yv) + b
        ex = jnp.exp(2.0 * xx)
        h3 = 1.0 - 2.0 * _rcp(ex + 1.0)
        plsc.store_scatter(hb2d, [w >> 1, w & 1], h3)

      @pl.when(sid < NS - 1)
      def _():
        pltpu.sync_copy(hb2d, h_hbm.at[pl.ds(nb, NSL)])

      @pl.when(sid == NS - 1)
      def _():
        pltpu.sync_copy(hb2d.at[pl.ds(0, N - (NS - 1) * NSL)],
                        h_hbm.at[pl.ds(nb, N - (NS - 1) * NSL)])

      @plsc.parallel_loop(0, NSL * 4 // L, unroll=2)
      def _lg(i):
        w4 = i * L + iota
        n = w4 >> 2
        j = w4 & 3
        s = plsc.load_gather(par_v, [j + 42])
        for k in range(2):
          hv = plsc.load_gather(hb2d, [n, jnp.full((L,), k, jnp.int32)])
          wv = plsc.load_gather(par_v, [34 + k * 4 + j])
          s = s + hv * wv
        y_sl[pl.ds(i * L, L)] = s

      @plsc.parallel_loop(0, NSL // L, unroll=2)
      def _ls(i):
        n = i * L + iota
        l0 = plsc.load_gather(y_sl, [n * 4])
        l1 = plsc.load_gather(y_sl, [n * 4 + 1])
        l2 = plsc.load_gather(y_sl, [n * 4 + 2])
        l3 = plsc.load_gather(y_sl, [n * 4 + 3])
        m = jnp.maximum(jnp.maximum(l0, l1), jnp.maximum(l2, l3))
        s = (jnp.exp(l0 - m) + jnp.exp(l1 - m) +
             jnp.exp(l2 - m) + jnp.exp(l3 - m))
        ibits = plsc.bitcast(s, jnp.int32)
        ev = ((ibits >> 23) & 0xFF) - 127
        mant = plsc.bitcast((ibits & 0x7FFFFF) | 0x3F800000, jnp.float32)
        t = (mant - 1.0) * _rcp(mant + 1.0)
        t2 = t * t
        lnm = 2.0 * t * (1.0 + t2 * (1.0 / 3.0 + t2 * (0.2 + t2 * (
            1.0 / 7.0 + t2 * (1.0 / 9.0)))))
        lse = m + ev.astype(jnp.float32) * 0.6931471805599453 + lnm
        plsc.store_scatter(deg2d, [n >> 3, n & 7], lse)

      @plsc.parallel_loop(0, NSL * 4 // L, unroll=2)
      def _out(i):
        w4 = i * L + iota
        l = y_sl[pl.ds(i * L, L)]
        n4 = w4 >> 2
        ls = plsc.load_gather(deg2d, [n4 >> 3, n4 & 7])
        plsc.store_scatter(ob2d, [n4, w4 & 3], l - ls)

      @pl.when(sid < NS - 1)
      def _():
        pltpu.sync_copy(ob2d, out_hbm.at[pl.ds(nb, NSL)])

      @pl.when(sid == NS - 1)
      def _():
        pltpu.sync_copy(ob2d.at[pl.ds(0, N - (NS - 1) * NSL)],
                        out_hbm.at[pl.ds(nb, N - (NS - 1) * NSL)])


def _xw1_body(x_ref, w1_ref, o_ref):
  o_ref[:N, :] = jnp.dot(x_ref[...], w1_ref[...],
                         preferred_element_type=jnp.float32)
  o_ref[N:, :] = jnp.zeros((NPAD - N, 4), jnp.float32)


_xw1 = pl.pallas_call(
    _xw1_body, out_shape=jax.ShapeDtypeStruct((NPAD, 4), jnp.float32))


@jax.jit
def kernel(x, edge_index, W1, b1, W2, b2, W3, b3, Wc, bc):
  ei = edge_index.astype(jnp.int32)
  idx = jnp.arange(ROWS4, dtype=jnp.int32).reshape(ROWS4 // RCH, RCH)
  par = jnp.concatenate([
      W2.reshape(-1), W3.reshape(-1), b1.reshape(-1), b2.reshape(-1),
      b3.reshape(-1), Wc.reshape(-1), bc.reshape(-1),
      jnp.zeros((2,), jnp.float32)]).astype(jnp.float32)

  xw1 = _xw1(x, W1)
  out, h = _sc_mega(xw1, ei, idx, par)
  return out, h
